# baseline (speedup 1.0000x reference)
import functools

import jax
import jax.numpy as jnp
from jax import lax
from jax.experimental import pallas as pl
from jax.experimental.pallas import tpu as pltpu
from jax.experimental.pallas import tpu_sc as plsc

NT = 16384
DM = 4096
NE = 64
NCORE = 2
NSUB = 16
NW = NCORE * NSUB
CH = NT // NW
BL = NT // NSUB
LSW = BL // 16
KR = 4
NCH = CH // KR

_mesh = plsc.VectorSubcoreMesh(
    core_axis_name="c", subcore_axis_name="s",
    num_cores=NCORE, num_subcores=NSUB)

_cparams = pltpu.CompilerParams(needs_layout_passes=False)


@functools.partial(
    pl.kernel,
    out_type=(jax.ShapeDtypeStruct((NT, DM), jnp.float32),
              jax.ShapeDtypeStruct((NE,), jnp.int32),
              jax.ShapeDtypeStruct((NT,), jnp.int32)),
    mesh=_mesh,
    compiler_params=_cparams,
    scratch_types=[
        pltpu.VMEM((BL,), jnp.int32),
        pltpu.VMEM((NE * 16,), jnp.int32),
        pltpu.VMEM((BL,), jnp.int32),
        pltpu.VMEM((NE * 16,), jnp.int32),
        pltpu.VMEM((NE * 16,), jnp.int32),
        pltpu.VMEM((NE,), jnp.int32),
        pltpu.VMEM((NSUB * NE,), jnp.int32),
        pltpu.VMEM((NE,), jnp.int32),
        pltpu.VMEM((NE,), jnp.int32),
        pltpu.VMEM((BL,), jnp.int32),
        pltpu.VMEM((CH,), jnp.int32),
        pltpu.VMEM((NCH, KR), jnp.int32),
        pltpu.VMEM((KR, DM), jnp.float32),
        pltpu.VMEM((KR, DM), jnp.float32),
        pltpu.VMEM((KR, DM), jnp.float32),
        pltpu.VMEM((KR, DM), jnp.float32),
        pltpu.VMEM_SHARED((NSUB * NE,), jnp.int32),
        pltpu.VMEM_SHARED((NT,), jnp.int32),
        pltpu.SemaphoreType.DMA,
        pltpu.SemaphoreType.DMA,
        pltpu.SemaphoreType.DMA,
        pltpu.SemaphoreType.DMA,
        pltpu.SemaphoreType.DMA,
        pltpu.SemaphoreType.DMA,
        pltpu.SemaphoreType.DMA,
        pltpu.SemaphoreType.DMA,
    ],
)
def _hardgate_kernel(in_hbm, map_hbm,
                     out_hbm, splits_hbm, rev_hbm,
                     mapb_v, cnt_v, rank_v, lb_v, incl_v, hist_v, ha_v,
                     tot_v, bb_v, revb_v, revw_v, rev2d,
                     rows_a, rows_b, rows_c, rows_d, sh_hist, sh_rev,
                     rsem_a, rsem_b, rsem_c, rsem_d,
                     wsem_a, wsem_b, wsem_c, wsem_d):
    s_loc = lax.axis_index("s")
    wid = lax.axis_index("c") * NSUB + s_loc
    tok0 = wid * CH
    blk0 = s_loc * BL

    bufs = ((rows_a, rsem_a, wsem_a),
            (rows_b, rsem_b, wsem_b),
            (rows_c, rsem_c, wsem_c),
            (rows_d, rsem_d, wsem_d))

    def _rd(c, buf, sem):
        return pltpu.make_async_copy(
            in_hbm.at[pl.ds(tok0 + c * KR, KR)], buf, sem)

    def _wr(c, buf, sem):
        return pltpu.make_async_copy(buf, out_hbm.at[rev2d.at[c]], sem)

    for b, (buf, rs, _) in enumerate(bufs):
        _rd(b, buf, rs).start()

    pltpu.sync_copy(map_hbm.at[pl.ds(blk0, BL)], mapb_v)
    lane = lax.iota(jnp.int32, 16)
    zeros = jnp.zeros((16,), jnp.int32)

    def zero_body(e, cr):
        cnt_v[pl.ds(e * 16, 16)] = zeros
        return cr
    lax.fori_loop(0, NE, zero_body, 0)

    def walk_body(st, cr):
        idx = lane * LSW + st
        ids = plsc.load_gather(mapb_v, [idx])
        key = ids * 16 + lane
        c = plsc.load_gather(cnt_v, [key])
        plsc.store_scatter(rank_v, [idx], c)
        plsc.store_scatter(cnt_v, [key], c + 1)
        return cr
    lax.fori_loop(0, LSW, walk_body, 0)

    def scan_body(e, cr):
        vec = cnt_v[pl.ds(e * 16, 16)]
        inc = plsc.cumsum(vec)
        lb_v[pl.ds(e * 16, 16)] = inc - vec
        incl_v[pl.ds(e * 16, 16)] = inc
        return cr
    lax.fori_loop(0, NE, scan_body, 0)

    for g in range(NE // 16):
        hv = plsc.load_gather(incl_v, [(g * 16 + lane) * 16 + 15])
        hist_v[pl.ds(g * 16, 16)] = hv

    pltpu.sync_copy(hist_v, sh_hist.at[pl.ds(s_loc * NE, NE)])
    plsc.subcore_barrier()
    pltpu.sync_copy(sh_hist, ha_v)

    for g in range(NE // 16):
        tot = jnp.zeros((16,), jnp.int32)
        pref = jnp.zeros((16,), jnp.int32)
        for w in range(NSUB):
            vec = ha_v[pl.ds(w * NE + g * 16, 16)]
            tot = tot + vec
            pref = pref + jnp.where(w < s_loc, vec, zeros)
        tot_v[pl.ds(g * 16, 16)] = tot
        bb_v[pl.ds(g * 16, 16)] = pref
    carry = jnp.int32(0)
    for g in range(NE // 16):
        vec = tot_v[pl.ds(g * 16, 16)]
        inc = plsc.cumsum(vec)
        bb_v[pl.ds(g * 16, 16)] = bb_v[pl.ds(g * 16, 16)] + (inc - vec) + carry
        carry = carry + jnp.sum(vec)

    @pl.when(wid == 0)
    def _():
        pltpu.sync_copy(tot_v, splits_hbm)

    def rev_body(v, cr):
        p = v * 16 + lane
        ids = mapb_v[pl.ds(v * 16, 16)]
        lane_of = p // LSW
        r = (rank_v[pl.ds(v * 16, 16)]
             + plsc.load_gather(lb_v, [ids * 16 + lane_of])
             + plsc.load_gather(bb_v, [ids]))
        revb_v[pl.ds(v * 16, 16)] = r
        return cr
    lax.fori_loop(0, BL // 16, rev_body, 0)

    pltpu.sync_copy(revb_v, sh_rev.at[pl.ds(blk0, BL)])
    plsc.subcore_barrier()
    pltpu.sync_copy(sh_rev.at[pl.ds(tok0, CH)], revw_v)
    pltpu.sync_copy(revw_v, rev_hbm.at[pl.ds(tok0, CH)])

    def rev2d_body(v, cr):
        t = v * 16 + lane
        r = revw_v[pl.ds(v * 16, 16)]
        plsc.store_scatter(rev2d, [t // KR, t % KR], r)
        return cr
    lax.fori_loop(0, CH // 16, rev2d_body, 0)

    NB = len(bufs)
    NTRIP = NCH // NB

    def ring_body(i, cr):
        cs = [i * NB + b for b in range(NB)]
        for ck, (buf, rs, ws) in zip(cs, bufs):
            _rd(ck, buf, rs).wait()
            _wr(ck, buf, ws).start()
        for ck, (buf, rs, ws) in zip(cs, bufs):
            _wr(ck, buf, ws).wait()

            @pl.when(ck + NB < NCH)
            def _():
                _rd(ck + NB, buf, rs).start()
        return cr

    lax.fori_loop(0, NTRIP, ring_body, 0)


def kernel(inputs, mapping):
    m32 = mapping.astype(jnp.int32)
    permuted, splits, rev = _hardgate_kernel(inputs, m32)
    return permuted, splits, rev


# speedup vs baseline: 1.0250x; 1.0250x over previous
import functools

import jax
import jax.numpy as jnp
from jax import lax
from jax.experimental import pallas as pl
from jax.experimental.pallas import tpu as pltpu
from jax.experimental.pallas import tpu_sc as plsc

NT = 16384
DM = 4096
NE = 64
NCORE = 2
NSUB = 16
NW = NCORE * NSUB
CH = NT // NW
BL = NT // NSUB
LSW = BL // 16
KR = 8
NCH = CH // KR

_mesh = plsc.VectorSubcoreMesh(
    core_axis_name="c", subcore_axis_name="s",
    num_cores=NCORE, num_subcores=NSUB)

_cparams = pltpu.CompilerParams(needs_layout_passes=False)


@functools.partial(
    pl.kernel,
    out_type=(jax.ShapeDtypeStruct((NT, DM), jnp.float32),
              jax.ShapeDtypeStruct((NE,), jnp.int32),
              jax.ShapeDtypeStruct((NT,), jnp.int32)),
    mesh=_mesh,
    compiler_params=_cparams,
    scratch_types=[
        pltpu.VMEM((BL,), jnp.int32),
        pltpu.VMEM((NE * 16,), jnp.int32),
        pltpu.VMEM((BL,), jnp.int32),
        pltpu.VMEM((NE * 16,), jnp.int32),
        pltpu.VMEM((NE * 16,), jnp.int32),
        pltpu.VMEM((NE,), jnp.int32),
        pltpu.VMEM((NSUB * NE,), jnp.int32),
        pltpu.VMEM((NE,), jnp.int32),
        pltpu.VMEM((NE,), jnp.int32),
        pltpu.VMEM((BL,), jnp.int32),
        pltpu.VMEM((NCH, KR), jnp.int32),
        pltpu.VMEM((KR, DM), jnp.float32),
        pltpu.VMEM((KR, DM), jnp.float32),
        pltpu.VMEM((KR, DM), jnp.float32),
        pltpu.VMEM_SHARED((NSUB * NE,), jnp.int32),
        pltpu.SemaphoreType.DMA,
        pltpu.SemaphoreType.DMA,
        pltpu.SemaphoreType.DMA,
        pltpu.SemaphoreType.DMA,
        pltpu.SemaphoreType.DMA,
        pltpu.SemaphoreType.DMA,
    ],
)
def _hardgate_kernel(in_hbm, map_hbm,
                     out_hbm, splits_hbm, rev_hbm,
                     mapb_v, cnt_v, rank_v, lb_v, incl_v, hist_v, ha_v,
                     tot_v, bb_v, revb_v, rev2d,
                     rows_a, rows_b, rows_c, sh_hist,
                     rsem_a, rsem_b, rsem_c, wsem_a, wsem_b, wsem_c):
    s_loc = lax.axis_index("s")
    cid = lax.axis_index("c")
    wid = cid * NSUB + s_loc
    blk0 = s_loc * BL
    coff = cid * CH
    tok0 = blk0 + coff

    bufs = ((rows_a, rsem_a, wsem_a),
            (rows_b, rsem_b, wsem_b),
            (rows_c, rsem_c, wsem_c))

    def _rd(c, buf, sem):
        return pltpu.make_async_copy(
            in_hbm.at[pl.ds(tok0 + c * KR, KR)], buf, sem)

    def _wr(c, buf, sem):
        return pltpu.make_async_copy(buf, out_hbm.at[rev2d.at[c]], sem)

    for b, (buf, rs, _) in enumerate(bufs):
        _rd(b, buf, rs).start()

    pltpu.sync_copy(map_hbm.at[pl.ds(blk0, BL)], mapb_v)
    lane = lax.iota(jnp.int32, 16)
    zeros = jnp.zeros((16,), jnp.int32)

    def zero_body(e, cr):
        cnt_v[pl.ds(e * 16, 16)] = zeros
        return cr
    lax.fori_loop(0, NE, zero_body, 0)

    def walk_body(st, cr):
        idx = lane * LSW + st
        ids = plsc.load_gather(mapb_v, [idx])
        key = ids * 16 + lane
        c = plsc.load_gather(cnt_v, [key])
        plsc.store_scatter(rank_v, [idx], c)
        plsc.store_scatter(cnt_v, [key], c + 1)
        return cr
    lax.fori_loop(0, LSW, walk_body, 0)

    def scan_body(e, cr):
        vec = cnt_v[pl.ds(e * 16, 16)]
        inc = plsc.cumsum(vec)
        lb_v[pl.ds(e * 16, 16)] = inc - vec
        incl_v[pl.ds(e * 16, 16)] = inc
        return cr
    lax.fori_loop(0, NE, scan_body, 0)

    for g in range(NE // 16):
        hv = plsc.load_gather(incl_v, [(g * 16 + lane) * 16 + 15])
        hist_v[pl.ds(g * 16, 16)] = hv

    pltpu.sync_copy(hist_v, sh_hist.at[pl.ds(s_loc * NE, NE)])
    plsc.subcore_barrier()
    pltpu.sync_copy(sh_hist, ha_v)

    for g in range(NE // 16):
        tot = jnp.zeros((16,), jnp.int32)
        pref = jnp.zeros((16,), jnp.int32)
        for w in range(NSUB):
            vec = ha_v[pl.ds(w * NE + g * 16, 16)]
            tot = tot + vec
            pref = pref + jnp.where(w < s_loc, vec, zeros)
        tot_v[pl.ds(g * 16, 16)] = tot
        bb_v[pl.ds(g * 16, 16)] = pref
    carry = jnp.int32(0)
    for g in range(NE // 16):
        vec = tot_v[pl.ds(g * 16, 16)]
        inc = plsc.cumsum(vec)
        bb_v[pl.ds(g * 16, 16)] = bb_v[pl.ds(g * 16, 16)] + (inc - vec) + carry
        carry = carry + jnp.sum(vec)

    @pl.when(wid == 0)
    def _():
        pltpu.sync_copy(tot_v, splits_hbm)

    def rev_body(v, cr):
        p = v * 16 + lane
        ids = mapb_v[pl.ds(v * 16, 16)]
        lane_of = p // LSW
        r = (rank_v[pl.ds(v * 16, 16)]
             + plsc.load_gather(lb_v, [ids * 16 + lane_of])
             + plsc.load_gather(bb_v, [ids]))
        revb_v[pl.ds(v * 16, 16)] = r
        return cr
    lax.fori_loop(0, BL // 16, rev_body, 0)

    pltpu.sync_copy(revb_v.at[pl.ds(coff, CH)], rev_hbm.at[pl.ds(tok0, CH)])

    def rev2d_body(v, cr):
        t = v * 16 + lane
        r = revb_v[pl.ds(coff + v * 16, 16)]
        plsc.store_scatter(rev2d, [t // KR, t % KR], r)
        return cr
    lax.fori_loop(0, CH // 16, rev2d_body, 0)

    NTRIP = NCH // 3

    def ring_body(i, cr):
        cs = [i * 3 + b for b in range(3)]
        for ck, (buf, rs, ws) in zip(cs, bufs):
            _rd(ck, buf, rs).wait()
            _wr(ck, buf, ws).start()
        for ck, (buf, rs, ws) in zip(cs, bufs):
            _wr(ck, buf, ws).wait()

            @pl.when(ck + 3 < NCH)
            def _():
                _rd(ck + 3, buf, rs).start()
        return cr

    lax.fori_loop(0, NTRIP, ring_body, 0)
    buf, rs, ws = bufs[0]
    _rd(NCH - 1, buf, rs).wait()
    _wr(NCH - 1, buf, ws).start()
    _wr(NCH - 1, buf, ws).wait()


def kernel(inputs, mapping):
    m32 = mapping.astype(jnp.int32)
    permuted, splits, rev = _hardgate_kernel(inputs, m32)
    return permuted, splits, rev
